# baseline (speedup 1.0000x reference)
import functools

import jax
import jax.numpy as jnp
from jax import lax
from jax.experimental import pallas as pl
from jax.experimental.pallas import tpu as pltpu
from jax.experimental.pallas import tpu_sc as plsc

N = 10000
E = 320000
D = 128
G = 128
C = 10

NC = 2
NS = 16
NW = NC * NS

CHUNK = 128
CHUNKS_PAD = 2560
E_PAD = CHUNKS_PAD * CHUNK
PH = 8
TPC0 = 56
TPC1 = 104
SPLIT = NS * TPC0
TPC2 = CHUNKS_PAD // NW
RPT = 640
N_ACC = NS * RPT
NBLOCK = RPT // CHUNK

_sc_mesh = plsc.VectorSubcoreMesh(
    core_axis_name="c", subcore_axis_name="s", num_cores=NC, num_subcores=NS)


@functools.partial(
    pl.kernel,
    out_type=[
        jax.ShapeDtypeStruct((NC * N_ACC, D), jnp.float32),
        jax.ShapeDtypeStruct((NC * N_ACC, D), jnp.float32),
    ],
    mesh=_sc_mesh,
    compiler_params=pltpu.CompilerParams(use_tc_tiling_on_sc=False),
    scratch_types=[
        pltpu.VMEM((PH, CHUNK), jnp.int32),
        pltpu.VMEM((PH, CHUNK), jnp.int32),
        pltpu.VMEM((CHUNK, D), jnp.float32),
        pltpu.VMEM_SHARED((N_ACC, D), jnp.float32),
        pltpu.SemaphoreType.DMA,
        pltpu.SemaphoreType.DMA,
        pltpu.SemaphoreType.DMA,
    ],
)
def _sc_agg(src_hbm, dst_hbm, x_hbm, ones_hbm, z_hbm,
            agg_out, deg_out,
            idx_s, idx_d, rows_v, acc_sh,
            semg, sema, semb):
    c = lax.axis_index("c")
    s = lax.axis_index("s")
    wid = c * NS + s
    out_base = c * N_ACC + s * RPT

    pltpu.sync_copy(z_hbm, rows_v)
    for j in range(NBLOCK):
        pltpu.sync_copy(rows_v, acc_sh.at[pl.ds(s * RPT + j * CHUNK, CHUNK)])
    plsc.subcore_barrier()

    base1 = jnp.where(c == 1, s * TPC0, SPLIT + s * TPC1)
    nph1 = jnp.where(c == 1, TPC0 // PH, TPC1 // PH)

    def phase1(p, carry):
        sl = pl.ds(base1 + p * PH, PH)
        pltpu.sync_copy(src_hbm.at[sl], idx_s)
        pltpu.sync_copy(dst_hbm.at[sl], idx_d)

        def body(t, carry2):
            pltpu.async_copy(x_hbm.at[idx_s.at[t]], rows_v, semg).wait()
            pltpu.async_copy(rows_v, acc_sh.at[idx_d.at[t]], sema,
                             add=True).wait()
            return carry2

        lax.fori_loop(0, PH, body, 0)
        return carry

    lax.fori_loop(0, nph1, phase1, 0)
    plsc.subcore_barrier()

    for j in range(NBLOCK):
        roff = s * RPT + j * CHUNK
        pltpu.sync_copy(acc_sh.at[pl.ds(roff, CHUNK)], rows_v)
        pltpu.sync_copy(rows_v, agg_out.at[pl.ds(out_base + j * CHUNK, CHUNK)])
    pltpu.sync_copy(z_hbm, rows_v)
    for j in range(NBLOCK):
        pltpu.sync_copy(rows_v, acc_sh.at[pl.ds(s * RPT + j * CHUNK, CHUNK)])
    pltpu.sync_copy(ones_hbm, rows_v)
    plsc.subcore_barrier()

    base2 = wid * TPC2

    def phase2(p, carry):
        pltpu.sync_copy(dst_hbm.at[pl.ds(base2 + p * PH, PH)], idx_d)

        def body2(j, carry2):
            s0 = pltpu.async_copy(rows_v, acc_sh.at[idx_d.at[2 * j]], sema,
                                  add=True)
            s1 = pltpu.async_copy(rows_v, acc_sh.at[idx_d.at[2 * j + 1]], semb,
                                  add=True)
            s0.wait()
            s1.wait()
            return carry2

        lax.fori_loop(0, PH // 2, body2, 0)
        return carry

    lax.fori_loop(0, TPC2 // PH, phase2, 0)
    plsc.subcore_barrier()

    for j in range(NBLOCK):
        roff = s * RPT + j * CHUNK
        pltpu.sync_copy(acc_sh.at[pl.ds(roff, CHUNK)], rows_v)
        pltpu.sync_copy(rows_v, deg_out.at[pl.ds(out_base + j * CHUNK, CHUNK)])


RB = 400
NBLK = N // RB


def _tc_post_body(agg_ref, deg_ref, batch_ref, w1_ref, b1_ref, w2_ref, b2_ref,
                  out_ref, pooled_ref):
    i = pl.program_id(0)

    agg = agg_ref[0] + agg_ref[1]
    deg = deg_ref[0, :, 0:1] + deg_ref[1, :, 0:1]
    xm = agg / jnp.maximum(deg, 1.0)
    h = jnp.dot(xm, w1_ref[...], preferred_element_type=jnp.float32)
    h = jnp.maximum(h + b1_ref[...], 0.0)

    b = batch_ref[0]
    gids = lax.broadcasted_iota(jnp.int32, (G, 1), 0)
    oh = (b == gids).astype(jnp.float32)

    @pl.when(i == 0)
    def _():
        pooled_ref[...] = jnp.zeros_like(pooled_ref)

    pooled_ref[...] += jnp.dot(oh, h, preferred_element_type=jnp.float32)

    @pl.when(i == NBLK - 1)
    def _():
        out_ref[...] = (
            jnp.dot(pooled_ref[...], w2_ref[...],
                    preferred_element_type=jnp.float32) + b2_ref[...])


_tc_post = pl.pallas_call(
    _tc_post_body,
    grid=(NBLK,),
    in_specs=[
        pl.BlockSpec((NC, RB, D), lambda i: (0, i, 0)),
        pl.BlockSpec((NC, RB, D), lambda i: (0, i, 0)),
        pl.BlockSpec((1, 1, RB), lambda i: (i, 0, 0)),
        pl.BlockSpec((D, D), lambda i: (0, 0)),
        pl.BlockSpec((1, D), lambda i: (0, 0)),
        pl.BlockSpec((D, C), lambda i: (0, 0)),
        pl.BlockSpec((1, C), lambda i: (0, 0)),
    ],
    out_specs=pl.BlockSpec((G, C), lambda i: (0, 0)),
    out_shape=jax.ShapeDtypeStruct((G, C), jnp.float32),
    scratch_shapes=[pltpu.VMEM((G, D), jnp.float32)],
)


@jax.jit
def kernel(x, edge_index, batch, W1, b1, W2, b2):
    npad = E_PAD - E
    src2d = jnp.concatenate(
        [edge_index[0], jnp.zeros((npad,), jnp.int32)]).reshape(CHUNKS_PAD, CHUNK)
    dst2d = jnp.concatenate(
        [edge_index[1], jnp.full((npad,), N, jnp.int32)]).reshape(CHUNKS_PAD, CHUNK)
    ones = jnp.ones((CHUNK, D), jnp.float32)
    z128 = jnp.zeros((CHUNK, D), jnp.float32)
    agg2, deg2 = _sc_agg(src2d, dst2d, x, ones, z128)
    agg3 = agg2.reshape(NC, N_ACC, D)
    deg3 = deg2.reshape(NC, N_ACC, D)
    batch3d = batch.reshape(NBLK, 1, RB)
    return _tc_post(agg3, deg3, batch3d, W1, b1.reshape(1, D),
                    W2, b2.reshape(1, C))


# speedup vs baseline: 1.0278x; 1.0278x over previous
import functools

import numpy as np

import jax
import jax.numpy as jnp
from jax import lax
from jax.experimental import pallas as pl
from jax.experimental.pallas import tpu as pltpu
from jax.experimental.pallas import tpu_sc as plsc

N = 10000
E = 320000
D = 128
G = 128
C = 10
DW = D // 2

NC = 2
NS = 16
NW = NC * NS

CHUNK = 128
TPC = 80
CHUNKS_PAD = NW * TPC
E_PAD = CHUNKS_PAD * CHUNK
PH = 8
NPH = TPC // PH
RPT = 640
N_ACC = NS * RPT
NBLOCK = RPT // CHUNK

_PERM = np.concatenate(
    [np.concatenate([g * 32 + 2 * np.arange(16),
                     g * 32 + 2 * np.arange(16) + 1])
     for g in range(D // 32)]).astype(np.int32)

_sc_mesh = plsc.VectorSubcoreMesh(
    core_axis_name="c", subcore_axis_name="s", num_cores=NC, num_subcores=NS)


@functools.partial(
    pl.kernel,
    out_type=[
        jax.ShapeDtypeStruct((NC * N_ACC, D), jnp.float32),
        jax.ShapeDtypeStruct((NC * N_ACC, D), jnp.float32),
    ],
    mesh=_sc_mesh,
    compiler_params=pltpu.CompilerParams(
        use_tc_tiling_on_sc=False, needs_layout_passes=False),
    scratch_types=[
        pltpu.VMEM((PH, CHUNK), jnp.int32),
        pltpu.VMEM((PH, CHUNK), jnp.int32),
        pltpu.VMEM((CHUNK, DW), jnp.int32),
        pltpu.VMEM((CHUNK, D), jnp.float32),
        pltpu.VMEM_SHARED((N_ACC, D), jnp.float32),
        pltpu.SemaphoreType.DMA,
        pltpu.SemaphoreType.DMA,
        pltpu.SemaphoreType.DMA,
    ],
)
def _sc_agg(src_hbm, dst_hbm, xp_hbm, ones_hbm, z_hbm,
            agg_out, deg_out,
            idx_s, idx_d, pk_v, rows_v, acc_sh,
            semg, sema, semb):
    c = lax.axis_index("c")
    s = lax.axis_index("s")
    wid = c * NS + s
    start = wid * TPC
    out_base = c * N_ACC + s * RPT

    pltpu.sync_copy(z_hbm, rows_v)
    for j in range(NBLOCK):
        pltpu.sync_copy(rows_v, acc_sh.at[pl.ds(s * RPT + j * CHUNK, CHUNK)])
    plsc.subcore_barrier()

    def phase1(p, carry):
        sl = pl.ds(start + p * PH, PH)
        pltpu.sync_copy(src_hbm.at[sl], idx_s)
        pltpu.sync_copy(dst_hbm.at[sl], idx_d)

        def body(t, carry2):
            pltpu.async_copy(xp_hbm.at[idx_s.at[t]], pk_v, semg).wait()

            def expand(i, carry3):
                r = i >> 2
                g = i & 3
                w = pk_v[r, pl.ds(g * 16, 16)]
                lo = plsc.bitcast(w << 16, jnp.float32)
                hi = plsc.bitcast(w & jnp.int32(-65536), jnp.float32)
                rows_v[r, pl.ds(g * 32, 16)] = lo
                rows_v[r, pl.ds(g * 32 + 16, 16)] = hi
                return carry3

            lax.fori_loop(0, CHUNK * 4, expand, 0)
            pltpu.async_copy(rows_v, acc_sh.at[idx_d.at[t]], sema,
                             add=True).wait()
            return carry2

        lax.fori_loop(0, PH, body, 0)
        return carry

    lax.fori_loop(0, NPH, phase1, 0)
    plsc.subcore_barrier()

    for j in range(NBLOCK):
        roff = s * RPT + j * CHUNK
        pltpu.sync_copy(acc_sh.at[pl.ds(roff, CHUNK)], rows_v)
        pltpu.sync_copy(rows_v, agg_out.at[pl.ds(out_base + j * CHUNK, CHUNK)])
    pltpu.sync_copy(z_hbm, rows_v)
    for j in range(NBLOCK):
        pltpu.sync_copy(rows_v, acc_sh.at[pl.ds(s * RPT + j * CHUNK, CHUNK)])
    pltpu.sync_copy(ones_hbm, rows_v)
    plsc.subcore_barrier()

    def phase2(p, carry):
        pltpu.sync_copy(dst_hbm.at[pl.ds(start + p * PH, PH)], idx_d)

        def body2(j, carry2):
            s0 = pltpu.async_copy(rows_v, acc_sh.at[idx_d.at[2 * j]], sema,
                                  add=True)
            s1 = pltpu.async_copy(rows_v, acc_sh.at[idx_d.at[2 * j + 1]], semb,
                                  add=True)
            s0.wait()
            s1.wait()
            return carry2

        lax.fori_loop(0, PH // 2, body2, 0)
        return carry

    lax.fori_loop(0, NPH, phase2, 0)
    plsc.subcore_barrier()

    for j in range(NBLOCK):
        roff = s * RPT + j * CHUNK
        pltpu.sync_copy(acc_sh.at[pl.ds(roff, CHUNK)], rows_v)
        pltpu.sync_copy(rows_v, deg_out.at[pl.ds(out_base + j * CHUNK, CHUNK)])


RB = 400
NBLK = N // RB


def _tc_post_body(agg_ref, deg_ref, batch_ref, w1_ref, b1_ref, w2_ref, b2_ref,
                  out_ref, pooled_ref):
    i = pl.program_id(0)

    agg = agg_ref[0] + agg_ref[1]
    deg = deg_ref[0, :, 0:1] + deg_ref[1, :, 0:1]
    xm = agg / jnp.maximum(deg, 1.0)
    h = jnp.dot(xm, w1_ref[...], preferred_element_type=jnp.float32)
    h = jnp.maximum(h + b1_ref[...], 0.0)

    b = batch_ref[0]
    gids = lax.broadcasted_iota(jnp.int32, (G, 1), 0)
    oh = (b == gids).astype(jnp.float32)

    @pl.when(i == 0)
    def _():
        pooled_ref[...] = jnp.zeros_like(pooled_ref)

    pooled_ref[...] += jnp.dot(oh, h, preferred_element_type=jnp.float32)

    @pl.when(i == NBLK - 1)
    def _():
        out_ref[...] = (
            jnp.dot(pooled_ref[...], w2_ref[...],
                    preferred_element_type=jnp.float32) + b2_ref[...])


_tc_post = pl.pallas_call(
    _tc_post_body,
    grid=(NBLK,),
    in_specs=[
        pl.BlockSpec((NC, RB, D), lambda i: (0, i, 0)),
        pl.BlockSpec((NC, RB, D), lambda i: (0, i, 0)),
        pl.BlockSpec((1, 1, RB), lambda i: (i, 0, 0)),
        pl.BlockSpec((D, D), lambda i: (0, 0)),
        pl.BlockSpec((1, D), lambda i: (0, 0)),
        pl.BlockSpec((D, C), lambda i: (0, 0)),
        pl.BlockSpec((1, C), lambda i: (0, 0)),
    ],
    out_specs=pl.BlockSpec((G, C), lambda i: (0, 0)),
    out_shape=jax.ShapeDtypeStruct((G, C), jnp.float32),
    scratch_shapes=[pltpu.VMEM((G, D), jnp.float32)],
)


@jax.jit
def kernel(x, edge_index, batch, W1, b1, W2, b2):
    npad = E_PAD - E
    src2d = jnp.concatenate(
        [edge_index[0], jnp.zeros((npad,), jnp.int32)]).reshape(CHUNKS_PAD, CHUNK)
    dst2d = jnp.concatenate(
        [edge_index[1], jnp.full((npad,), N, jnp.int32)]).reshape(CHUNKS_PAD, CHUNK)
    xp = lax.bitcast_convert_type(
        x.astype(jnp.bfloat16).reshape(N, DW, 2), jnp.int32)
    ones = jnp.ones((CHUNK, D), jnp.float32)
    z128 = jnp.zeros((CHUNK, D), jnp.float32)
    agg2, deg2 = _sc_agg(src2d, dst2d, xp, ones, z128)
    agg3 = agg2.reshape(NC, N_ACC, D)
    deg3 = deg2.reshape(NC, N_ACC, D)
    batch3d = batch.reshape(NBLK, 1, RB)
    w1p = W1[jnp.asarray(_PERM), :]
    return _tc_post(agg3, deg3, batch3d, w1p, b1.reshape(1, D),
                    W2, b2.reshape(1, C))


# speedup vs baseline: 1.1269x; 1.0964x over previous
import functools

import jax
import jax.numpy as jnp
from jax import lax
from jax.experimental import pallas as pl
from jax.experimental.pallas import tpu as pltpu
from jax.experimental.pallas import tpu_sc as plsc

N = 10000
E = 320000
D = 128
G = 128
C = 10

NC = 2
NS = 16
NW = NC * NS

CHUNK = 64
TPC = 160
CHUNKS_PAD = NW * TPC
E_PAD = CHUNKS_PAD * CHUNK
PH = 32
NPH = TPC // PH
RPT = 640
N_ACC = NS * RPT
NBLOCK = RPT // CHUNK

_sc_mesh = plsc.VectorSubcoreMesh(
    core_axis_name="c", subcore_axis_name="s", num_cores=NC, num_subcores=NS)


@functools.partial(
    pl.kernel,
    out_type=[
        jax.ShapeDtypeStruct((NC * N_ACC, D), jnp.float32),
        jax.ShapeDtypeStruct((NC * N_ACC, 16), jnp.float32),
    ],
    mesh=_sc_mesh,
    compiler_params=pltpu.CompilerParams(use_tc_tiling_on_sc=False),
    scratch_types=[
        pltpu.VMEM((PH, CHUNK), jnp.int32),
        pltpu.VMEM((PH, CHUNK), jnp.int32),
        pltpu.VMEM((CHUNK, D), jnp.float32),
        pltpu.VMEM((CHUNK, 16), jnp.float32),
        pltpu.VMEM_SHARED((N_ACC, D), jnp.float32),
        pltpu.VMEM_SHARED((N_ACC, 16), jnp.float32),
        pltpu.SemaphoreType.DMA,
    ],
)
def _sc_agg(src_hbm, dst_hbm, x_hbm, ones_hbm, z128_hbm, z16_hbm,
            agg_out, deg_out,
            idx_s, idx_d, rows_v, ones_v, agg_sh, deg_sh, sem):
    c = lax.axis_index("c")
    s = lax.axis_index("s")
    wid = c * NS + s
    start = wid * TPC

    pltpu.sync_copy(z128_hbm, rows_v)
    pltpu.sync_copy(z16_hbm, ones_v)
    for j in range(NBLOCK):
        zsl = pl.ds(s * RPT + j * CHUNK, CHUNK)
        pltpu.sync_copy(rows_v, agg_sh.at[zsl])
        pltpu.sync_copy(ones_v, deg_sh.at[zsl])
    pltpu.sync_copy(ones_hbm, ones_v)
    plsc.subcore_barrier()

    for p in range(NPH):
        pltpu.sync_copy(src_hbm.at[pl.ds(start + p * PH, PH)], idx_s)
        pltpu.sync_copy(dst_hbm.at[pl.ds(start + p * PH, PH)], idx_d)

        def body(t, carry):
            pltpu.async_copy(x_hbm.at[idx_s.at[t]], rows_v, sem).wait()
            pltpu.sync_copy(rows_v, agg_sh.at[idx_d.at[t]], add=True)
            pltpu.sync_copy(ones_v, deg_sh.at[idx_d.at[t]], add=True)
            return carry

        lax.fori_loop(0, PH, body, 0)
    plsc.subcore_barrier()

    for j in range(NBLOCK):
        roff = s * RPT + j * CHUNK
        pltpu.sync_copy(agg_sh.at[pl.ds(roff, CHUNK)], rows_v)
        pltpu.sync_copy(rows_v, agg_out.at[pl.ds(c * N_ACC + roff, CHUNK)])
        pltpu.sync_copy(deg_sh.at[pl.ds(roff, CHUNK)], ones_v)
        pltpu.sync_copy(ones_v, deg_out.at[pl.ds(c * N_ACC + roff, CHUNK)])


RB = 400
NBLK = N // RB


def _tc_post_body(agg_ref, deg_ref, batch_ref, w1_ref, b1_ref, w2_ref, b2_ref,
                  out_ref, pooled_ref):
    i = pl.program_id(0)

    agg = agg_ref[0] + agg_ref[1]
    deg = deg_ref[0, :, 0:1] + deg_ref[1, :, 0:1]
    xm = agg / jnp.maximum(deg, 1.0)
    h = jnp.dot(xm, w1_ref[...], preferred_element_type=jnp.float32)
    h = jnp.maximum(h + b1_ref[...], 0.0)

    b = batch_ref[0]
    gids = lax.broadcasted_iota(jnp.int32, (G, 1), 0)
    oh = (b == gids).astype(jnp.float32)

    @pl.when(i == 0)
    def _():
        pooled_ref[...] = jnp.zeros_like(pooled_ref)

    pooled_ref[...] += jnp.dot(oh, h, preferred_element_type=jnp.float32)

    @pl.when(i == NBLK - 1)
    def _():
        out_ref[...] = (
            jnp.dot(pooled_ref[...], w2_ref[...],
                    preferred_element_type=jnp.float32) + b2_ref[...])


_tc_post = pl.pallas_call(
    _tc_post_body,
    grid=(NBLK,),
    in_specs=[
        pl.BlockSpec((NC, RB, D), lambda i: (0, i, 0)),
        pl.BlockSpec((NC, RB, 16), lambda i: (0, i, 0)),
        pl.BlockSpec((1, 1, RB), lambda i: (i, 0, 0)),
        pl.BlockSpec((D, D), lambda i: (0, 0)),
        pl.BlockSpec((1, D), lambda i: (0, 0)),
        pl.BlockSpec((D, C), lambda i: (0, 0)),
        pl.BlockSpec((1, C), lambda i: (0, 0)),
    ],
    out_specs=pl.BlockSpec((G, C), lambda i: (0, 0)),
    out_shape=jax.ShapeDtypeStruct((G, C), jnp.float32),
    scratch_shapes=[pltpu.VMEM((G, D), jnp.float32)],
)


@jax.jit
def kernel(x, edge_index, batch, W1, b1, W2, b2):
    npad = E_PAD - E
    src2d = jnp.concatenate(
        [edge_index[0], jnp.zeros((npad,), jnp.int32)]).reshape(CHUNKS_PAD, CHUNK)
    dst2d = jnp.concatenate(
        [edge_index[1], jnp.full((npad,), N, jnp.int32)]).reshape(CHUNKS_PAD, CHUNK)
    ones = jnp.ones((CHUNK, 16), jnp.float32)
    z128 = jnp.zeros((CHUNK, D), jnp.float32)
    z16 = jnp.zeros((CHUNK, 16), jnp.float32)
    agg2, deg2 = _sc_agg(src2d, dst2d, x, ones, z128, z16)
    agg3 = agg2.reshape(NC, N_ACC, D)
    deg3 = deg2.reshape(NC, N_ACC, 16)
    batch3d = batch.reshape(NBLK, 1, RB)
    return _tc_post(agg3, deg3, batch3d, W1, b1.reshape(1, D),
                    W2, b2.reshape(1, C))
